# baseline (speedup 1.0000x reference)
import jax
import jax.numpy as jnp
from jax import lax
from jax.experimental import pallas as pl
from jax.experimental.pallas import tpu as pltpu
from jax.experimental.pallas import tpu_sc as plsc

N_NODES = 10000
N_PAD = 10240
D = 128
DH = 64
N_EDGES = 320000
NW = 32
K = 128
NCH = 79
EPW = K * NCH
NE_PAD = NW * EPW
RPT = 640
DEG_W = 16

_mesh = plsc.VectorSubcoreMesh(core_axis_name="c", subcore_axis_name="s")


def _deg_body(dst_hbm, out_hbm, dstv, onesv, zbuf, shared):
    cid = lax.axis_index("c")
    sid = lax.axis_index("s")
    wid = sid * 2 + cid
    pltpu.sync_copy(dst_hbm.at[wid], dstv)

    def fill(i, _):
        zbuf[i, :] = jnp.zeros((16,), jnp.float32)
        onesv[i % K, :] = jnp.ones((16,), jnp.float32)
        return 0

    lax.fori_loop(0, RPT, fill, 0)
    pltpu.sync_copy(zbuf, shared.at[pl.ds(sid * RPT, RPT)])
    plsc.subcore_barrier()

    def chunk(j, _):
        pltpu.sync_copy(onesv, shared.at[dstv.at[j]], add=True)
        return 0

    lax.fori_loop(0, NCH, chunk, 0)
    plsc.subcore_barrier()
    pltpu.sync_copy(shared.at[pl.ds(sid * RPT, RPT)],
                    out_hbm.at[cid, pl.ds(sid * RPT, RPT)])


_deg_call = pl.kernel(
    _deg_body,
    out_type=jax.ShapeDtypeStruct((2, N_PAD, DEG_W), jnp.float32),
    mesh=_mesh,
    scratch_types=[
        pltpu.VMEM((NCH, K), jnp.int32),
        pltpu.VMEM((K, DEG_W), jnp.float32),
        pltpu.VMEM((RPT, DEG_W), jnp.float32),
        pltpu.VMEM_SHARED((N_PAD, DEG_W), jnp.float32),
    ],
    compiler_params=pltpu.CompilerParams(use_tc_tiling_on_sc=False),
)


def _edge_body(ya_hbm, yb_hbm, src_hbm, dst_hbm, out_hbm,
               srcv, dstv, rows, rows1, rows2, rows3, zbuf, shared,
               sem, sem1, sem2, sem3, ssem, ssem1, ssem2, ssem3):
    cid = lax.axis_index("c")
    sid = lax.axis_index("s")
    wid = sid * 2 + cid
    pltpu.sync_copy(src_hbm.at[wid], srcv)
    pltpu.sync_copy(dst_hbm.at[wid], dstv)

    def zrow(i, _):
        def zcol(j, _):
            zbuf[i, pl.ds(j * 16, 16)] = jnp.zeros((16,), jnp.float32)
            return 0
        lax.fori_loop(0, DH // 16, zcol, 0)
        return 0

    lax.fori_loop(0, K, zrow, 0)

    for half, y_hbm in ((0, ya_hbm.at[cid]), (1, yb_hbm.at[cid])):
        def zshared(k, _):
            pltpu.sync_copy(zbuf, shared.at[pl.ds(sid * RPT + k * K, K)])
            return 0

        lax.fori_loop(0, RPT // K, zshared, 0)
        plsc.subcore_barrier()

        bufs = ((rows, sem, ssem), (rows1, sem1, ssem1),
                (rows2, sem2, ssem2), (rows3, sem3, ssem3))
        nb = len(bufs)
        for b in range(nb):
            pltpu.async_copy(y_hbm.at[srcv.at[b]], bufs[b][0], bufs[b][1])

        def group(g, _):
            j = g * nb
            for b in range(nb):
                r, gs, ss = bufs[b]
                pltpu.make_async_copy(y_hbm.at[srcv.at[0]], r, gs).wait()
                pltpu.async_copy(r, shared.at[dstv.at[j + b]], ss, add=True)
            for b in range(nb):
                r, gs, ss = bufs[b]
                pltpu.make_async_copy(r, shared.at[dstv.at[0]], ss).wait()
                pltpu.async_copy(y_hbm.at[srcv.at[j + nb + b]], r, gs)
            return 0

        ngrp = (NCH - nb - 3) // nb
        lax.fori_loop(0, ngrp, group, 0)
        for b in range(nb):
            r, gs, ss = bufs[b]
            j = ngrp * nb + b
            pltpu.make_async_copy(y_hbm.at[srcv.at[0]], r, gs).wait()
            pltpu.async_copy(r, shared.at[dstv.at[j]], ss, add=True)
            if b < 3:
                pltpu.make_async_copy(r, shared.at[dstv.at[0]], ss).wait()
                pltpu.async_copy(y_hbm.at[srcv.at[j + nb]], r, gs)
        for b in range(3):
            r, gs, ss = bufs[b]
            j = ngrp * nb + nb + b
            pltpu.make_async_copy(y_hbm.at[srcv.at[0]], r, gs).wait()
            pltpu.async_copy(r, shared.at[dstv.at[j]], ss, add=True)
        for b in range(nb):
            r, gs, ss = bufs[b]
            pltpu.make_async_copy(r, shared.at[dstv.at[0]], ss).wait()
        plsc.subcore_barrier()
        pltpu.sync_copy(shared.at[pl.ds(sid * RPT, RPT)],
                        out_hbm.at[half, cid, pl.ds(sid * RPT, RPT)])


_edge_call = pl.kernel(
    _edge_body,
    out_type=jax.ShapeDtypeStruct((2, 2, N_PAD, DH), jnp.float32),
    mesh=_mesh,
    scratch_types=[
        pltpu.VMEM((NCH, K), jnp.int32),
        pltpu.VMEM((NCH, K), jnp.int32),
        pltpu.VMEM((K, DH), jnp.float32),
        pltpu.VMEM((K, DH), jnp.float32),
        pltpu.VMEM((K, DH), jnp.float32),
        pltpu.VMEM((K, DH), jnp.float32),
        pltpu.VMEM((K, DH), jnp.float32),
        pltpu.VMEM_SHARED((N_PAD, DH), jnp.float32),
        pltpu.SemaphoreType.DMA,
        pltpu.SemaphoreType.DMA,
        pltpu.SemaphoreType.DMA,
        pltpu.SemaphoreType.DMA,
        pltpu.SemaphoreType.DMA,
        pltpu.SemaphoreType.DMA,
        pltpu.SemaphoreType.DMA,
        pltpu.SemaphoreType.DMA,
    ],
    compiler_params=pltpu.CompilerParams(use_tc_tiling_on_sc=False),
)


_GRID = 10
_RB = N_NODES // _GRID


def _mm_body(x_ref, w_ref, o_ref):
    o_ref[...] = jnp.dot(x_ref[...], w_ref[...],
                         preferred_element_type=jnp.float32)


def _matmul(x, w):
    return pl.pallas_call(
        _mm_body,
        grid=(_GRID,),
        in_specs=[
            pl.BlockSpec((_RB, D), lambda i: (i, 0)),
            pl.BlockSpec((D, D), lambda i: (0, 0)),
        ],
        out_specs=pl.BlockSpec((_RB, D), lambda i: (i, 0)),
        out_shape=jax.ShapeDtypeStruct((N_NODES, D), jnp.float32),
    )(x, w)


def _scale_body(deg_ref, xw_ref, dinv_ref, ya_ref, yb_ref):
    dp = deg_ref[...]
    deg = 1.0 + dp[0, :, 0] + dp[1, :, 0]
    dinv = lax.rsqrt(deg)[:, None]
    dinv_ref[...] = dinv
    y = xw_ref[...] * dinv
    ya_ref[...] = jnp.broadcast_to(y[None, :, :DH], (2, _RB, DH))
    yb_ref[...] = jnp.broadcast_to(y[None, :, DH:], (2, _RB, DH))


def _scale(degp, xw):
    return pl.pallas_call(
        _scale_body,
        grid=(_GRID,),
        in_specs=[
            pl.BlockSpec((2, _RB, DEG_W), lambda i: (0, i, 0)),
            pl.BlockSpec((_RB, D), lambda i: (i, 0)),
        ],
        out_specs=[
            pl.BlockSpec((_RB, 1), lambda i: (i, 0)),
            pl.BlockSpec((2, _RB, DH), lambda i: (0, i, 0)),
            pl.BlockSpec((2, _RB, DH), lambda i: (0, i, 0)),
        ],
        out_shape=[
            jax.ShapeDtypeStruct((N_NODES, 1), jnp.float32),
            jax.ShapeDtypeStruct((2, N_NODES, DH), jnp.float32),
            jax.ShapeDtypeStruct((2, N_NODES, DH), jnp.float32),
        ],
    )(degp, xw)


def _mid_body(pa_ref, ya_ref, yb_ref, dinv_ref, b1_ref, w2_ref,
              y2a_ref, y2b_ref):
    pa = pa_ref[...]
    dinv = dinv_ref[...]
    acc = jnp.concatenate(
        [pa[0, 0] + pa[0, 1] + ya_ref[0],
         pa[1, 0] + pa[1, 1] + yb_ref[0]], axis=1)
    h = jnp.maximum(acc * dinv + b1_ref[...], 0.0)
    y2 = jnp.dot(h, w2_ref[...], preferred_element_type=jnp.float32) * dinv
    y2a_ref[...] = jnp.broadcast_to(y2[None, :, :DH], (2, _RB, DH))
    y2b_ref[...] = jnp.broadcast_to(y2[None, :, DH:], (2, _RB, DH))


def _mid(pa, ya, yb, dinv, b1, w2):
    return pl.pallas_call(
        _mid_body,
        grid=(_GRID,),
        in_specs=[
            pl.BlockSpec((2, 2, _RB, DH), lambda i: (0, 0, i, 0)),
            pl.BlockSpec((1, _RB, DH), lambda i: (0, i, 0)),
            pl.BlockSpec((1, _RB, DH), lambda i: (0, i, 0)),
            pl.BlockSpec((_RB, 1), lambda i: (i, 0)),
            pl.BlockSpec((1, D), lambda i: (0, 0)),
            pl.BlockSpec((D, D), lambda i: (0, 0)),
        ],
        out_specs=[
            pl.BlockSpec((2, _RB, DH), lambda i: (0, i, 0)),
            pl.BlockSpec((2, _RB, DH), lambda i: (0, i, 0)),
        ],
        out_shape=[
            jax.ShapeDtypeStruct((2, N_NODES, DH), jnp.float32),
            jax.ShapeDtypeStruct((2, N_NODES, DH), jnp.float32),
        ],
    )(pa, ya, yb, dinv, b1, w2)


def _final_body(pa_ref, y2a_ref, y2b_ref, dinv_ref, b2_ref, o_ref):
    pa = pa_ref[...]
    acc = jnp.concatenate(
        [pa[0, 0] + pa[0, 1] + y2a_ref[0],
         pa[1, 0] + pa[1, 1] + y2b_ref[0]], axis=1)
    o_ref[...] = acc * dinv_ref[...] + b2_ref[...]


def _final(pa, y2a, y2b, dinv, b2):
    return pl.pallas_call(
        _final_body,
        grid=(_GRID,),
        in_specs=[
            pl.BlockSpec((2, 2, _RB, DH), lambda i: (0, 0, i, 0)),
            pl.BlockSpec((1, _RB, DH), lambda i: (0, i, 0)),
            pl.BlockSpec((1, _RB, DH), lambda i: (0, i, 0)),
            pl.BlockSpec((_RB, 1), lambda i: (i, 0)),
            pl.BlockSpec((1, D), lambda i: (0, 0)),
        ],
        out_specs=pl.BlockSpec((_RB, D), lambda i: (i, 0)),
        out_shape=jax.ShapeDtypeStruct((N_NODES, D), jnp.float32),
    )(pa, y2a, y2b, dinv, b2)


def kernel(x, edge_index, W1, b1, W2, b2):
    ei = edge_index.astype(jnp.int32)
    pad = NE_PAD - N_EDGES
    src = jnp.concatenate([ei[0], jnp.zeros((pad,), jnp.int32)])
    junk = N_NODES + (jnp.arange(pad, dtype=jnp.int32) % (N_PAD - N_NODES))
    dst = jnp.concatenate([ei[1], junk])
    src_r = src.reshape(NW, NCH, K)
    dst_r = dst.reshape(NW, NCH, K)

    degp = _deg_call(dst_r)
    xw1 = _matmul(x, W1)
    dinv, y1a, y1b = _scale(degp, xw1)

    pa1 = _edge_call(y1a, y1b, src_r, dst_r)
    y2a, y2b = _mid(pa1, y1a, y1b, dinv, b1.reshape(1, D), W2)
    pa2 = _edge_call(y2a, y2b, src_r, dst_r)
    return _final(pa2, y2a, y2b, dinv, b2.reshape(1, D))


# speedup vs baseline: 1.1248x; 1.1248x over previous
import jax
import jax.numpy as jnp
from jax import lax
from jax.experimental import pallas as pl
from jax.experimental.pallas import tpu as pltpu
from jax.experimental.pallas import tpu_sc as plsc

N_NODES = 10000
N_PAD = 10240
D = 128
DH = 64
N_EDGES = 320000
NW = 32
K = 128
NCH = 79
EPW = K * NCH
NE_PAD = NW * EPW
RPT = 640
DEG_W = 16

_mesh = plsc.VectorSubcoreMesh(core_axis_name="c", subcore_axis_name="s")


def _deg_body(dst_hbm, out_hbm, dstv, onesv, zbuf, shared):
    cid = lax.axis_index("c")
    sid = lax.axis_index("s")
    wid = sid * 2 + cid
    pltpu.sync_copy(dst_hbm.at[wid], dstv)

    def fill(i, _):
        zbuf[i, :] = jnp.zeros((16,), jnp.float32)
        onesv[i % K, :] = jnp.ones((16,), jnp.float32)
        return 0

    lax.fori_loop(0, RPT, fill, 0)
    pltpu.sync_copy(zbuf, shared.at[pl.ds(sid * RPT, RPT)])
    plsc.subcore_barrier()

    def chunk(j, _):
        pltpu.sync_copy(onesv, shared.at[dstv.at[j]], add=True)
        return 0

    lax.fori_loop(0, NCH, chunk, 0)
    plsc.subcore_barrier()
    pltpu.sync_copy(shared.at[pl.ds(sid * RPT, RPT)],
                    out_hbm.at[cid, pl.ds(sid * RPT, RPT)])


_deg_call = pl.kernel(
    _deg_body,
    out_type=jax.ShapeDtypeStruct((2, N_PAD, DEG_W), jnp.float32),
    mesh=_mesh,
    scratch_types=[
        pltpu.VMEM((NCH, K), jnp.int32),
        pltpu.VMEM((K, DEG_W), jnp.float32),
        pltpu.VMEM((RPT, DEG_W), jnp.float32),
        pltpu.VMEM_SHARED((N_PAD, DEG_W), jnp.float32),
    ],
    compiler_params=pltpu.CompilerParams(use_tc_tiling_on_sc=False),
)


def _edge_body(ya_hbm, yb_hbm, src_hbm, dst_hbm, out_hbm,
               srcv, dstv, rows, rows1, rows2, rows3, zbuf, shared,
               sem, sem1, sem2, sem3, ssem, ssem1, ssem2, ssem3):
    cid = lax.axis_index("c")
    sid = lax.axis_index("s")
    wid = sid * 2 + cid
    pltpu.sync_copy(src_hbm.at[wid], srcv)
    pltpu.sync_copy(dst_hbm.at[wid], dstv)

    def zrow(i, _):
        def zcol(j, _):
            zbuf[i, pl.ds(j * 16, 16)] = jnp.zeros((16,), jnp.float32)
            return 0
        lax.fori_loop(0, DH // 16, zcol, 0)
        return 0

    lax.fori_loop(0, K, zrow, 0)

    bufs = ((rows, sem, ssem), (rows1, sem1, ssem1),
            (rows2, sem2, ssem2), (rows3, sem3, ssem3))
    nb = len(bufs)

    def zshared(k, _):
        pltpu.sync_copy(zbuf, shared.at[pl.ds(sid * RPT + k * K, K)])
        return 0

    lax.fori_loop(0, RPT // K, zshared, 0)
    plsc.subcore_barrier()
    for b in range(nb):
        pltpu.async_copy(ya_hbm.at[srcv.at[b]], bufs[b][0], bufs[b][1])

    for half, y_hbm in ((0, ya_hbm), (1, yb_hbm)):
        def group(g, _):
            j = g * nb
            for b in range(nb):
                r, gs, ss = bufs[b]
                pltpu.make_async_copy(y_hbm.at[srcv.at[0]], r, gs).wait()
                pltpu.async_copy(r, shared.at[dstv.at[j + b]], ss, add=True)
            for b in range(nb):
                r, gs, ss = bufs[b]
                pltpu.make_async_copy(r, shared.at[dstv.at[0]], ss).wait()
                pltpu.async_copy(y_hbm.at[srcv.at[j + nb + b]], r, gs)
            return 0

        ngrp = (NCH - nb - 3) // nb
        lax.fori_loop(0, ngrp, group, 0)
        for b in range(nb):
            r, gs, ss = bufs[b]
            j = ngrp * nb + b
            pltpu.make_async_copy(y_hbm.at[srcv.at[0]], r, gs).wait()
            pltpu.async_copy(r, shared.at[dstv.at[j]], ss, add=True)
            if b < 3:
                pltpu.make_async_copy(r, shared.at[dstv.at[0]], ss).wait()
                pltpu.async_copy(y_hbm.at[srcv.at[j + nb]], r, gs)
        for b in range(3):
            r, gs, ss = bufs[b]
            j = ngrp * nb + nb + b
            pltpu.make_async_copy(y_hbm.at[srcv.at[0]], r, gs).wait()
            pltpu.async_copy(r, shared.at[dstv.at[j]], ss, add=True)
        for b in range(nb):
            r, gs, ss = bufs[b]
            pltpu.make_async_copy(r, shared.at[dstv.at[0]], ss).wait()
        plsc.subcore_barrier()
        if half == 0:
            for b in range(nb):
                pltpu.async_copy(yb_hbm.at[srcv.at[b]], bufs[b][0],
                                 bufs[b][1])
        pltpu.sync_copy(shared.at[pl.ds(sid * RPT, RPT)],
                        out_hbm.at[half, cid, pl.ds(sid * RPT, RPT)])
        if half == 0:
            lax.fori_loop(0, RPT // K, zshared, 0)
            plsc.subcore_barrier()


_edge_call = pl.kernel(
    _edge_body,
    out_type=jax.ShapeDtypeStruct((2, 2, N_PAD, DH), jnp.float32),
    mesh=_mesh,
    scratch_types=[
        pltpu.VMEM((NCH, K), jnp.int32),
        pltpu.VMEM((NCH, K), jnp.int32),
        pltpu.VMEM((K, DH), jnp.float32),
        pltpu.VMEM((K, DH), jnp.float32),
        pltpu.VMEM((K, DH), jnp.float32),
        pltpu.VMEM((K, DH), jnp.float32),
        pltpu.VMEM((K, DH), jnp.float32),
        pltpu.VMEM_SHARED((N_PAD, DH), jnp.float32),
        pltpu.SemaphoreType.DMA,
        pltpu.SemaphoreType.DMA,
        pltpu.SemaphoreType.DMA,
        pltpu.SemaphoreType.DMA,
        pltpu.SemaphoreType.DMA,
        pltpu.SemaphoreType.DMA,
        pltpu.SemaphoreType.DMA,
        pltpu.SemaphoreType.DMA,
    ],
    compiler_params=pltpu.CompilerParams(use_tc_tiling_on_sc=False),
)


_GRID = 10
_RB = N_NODES // _GRID


def _mm_body(x_ref, w_ref, o_ref):
    o_ref[...] = jnp.dot(x_ref[...], w_ref[...],
                         preferred_element_type=jnp.float32)


def _matmul(x, w):
    return pl.pallas_call(
        _mm_body,
        grid=(_GRID,),
        in_specs=[
            pl.BlockSpec((_RB, D), lambda i: (i, 0)),
            pl.BlockSpec((D, D), lambda i: (0, 0)),
        ],
        out_specs=pl.BlockSpec((_RB, D), lambda i: (i, 0)),
        out_shape=jax.ShapeDtypeStruct((N_NODES, D), jnp.float32),
    )(x, w)


def _scale_body(deg_ref, x_ref, w_ref, dinv_ref, ya_ref, yb_ref):
    dp = deg_ref[...]
    deg = 1.0 + dp[0, :, 0] + dp[1, :, 0]
    dinv = lax.rsqrt(deg)[:, None]
    dinv_ref[...] = dinv
    y = jnp.dot(x_ref[...], w_ref[...],
                preferred_element_type=jnp.float32) * dinv
    ya_ref[...] = y[:, :DH]
    yb_ref[...] = y[:, DH:]


def _scale(degp, x, w):
    return pl.pallas_call(
        _scale_body,
        grid=(_GRID,),
        in_specs=[
            pl.BlockSpec((2, _RB, DEG_W), lambda i: (0, i, 0)),
            pl.BlockSpec((_RB, D), lambda i: (i, 0)),
            pl.BlockSpec((D, D), lambda i: (0, 0)),
        ],
        out_specs=[
            pl.BlockSpec((_RB, 1), lambda i: (i, 0)),
            pl.BlockSpec((_RB, DH), lambda i: (i, 0)),
            pl.BlockSpec((_RB, DH), lambda i: (i, 0)),
        ],
        out_shape=[
            jax.ShapeDtypeStruct((N_NODES, 1), jnp.float32),
            jax.ShapeDtypeStruct((N_NODES, DH), jnp.float32),
            jax.ShapeDtypeStruct((N_NODES, DH), jnp.float32),
        ],
    )(degp, x, w)


def _mid_body(pa_ref, ya_ref, yb_ref, dinv_ref, b1_ref, w2_ref,
              y2a_ref, y2b_ref):
    pa = pa_ref[...]
    dinv = dinv_ref[...]
    acc = jnp.concatenate(
        [pa[0, 0] + pa[0, 1] + ya_ref[...],
         pa[1, 0] + pa[1, 1] + yb_ref[...]], axis=1)
    h = jnp.maximum(acc * dinv + b1_ref[...], 0.0)
    y2 = jnp.dot(h, w2_ref[...], preferred_element_type=jnp.float32) * dinv
    y2a_ref[...] = y2[:, :DH]
    y2b_ref[...] = y2[:, DH:]


def _mid(pa, ya, yb, dinv, b1, w2):
    return pl.pallas_call(
        _mid_body,
        grid=(_GRID,),
        in_specs=[
            pl.BlockSpec((2, 2, _RB, DH), lambda i: (0, 0, i, 0)),
            pl.BlockSpec((_RB, DH), lambda i: (i, 0)),
            pl.BlockSpec((_RB, DH), lambda i: (i, 0)),
            pl.BlockSpec((_RB, 1), lambda i: (i, 0)),
            pl.BlockSpec((1, D), lambda i: (0, 0)),
            pl.BlockSpec((D, D), lambda i: (0, 0)),
        ],
        out_specs=[
            pl.BlockSpec((_RB, DH), lambda i: (i, 0)),
            pl.BlockSpec((_RB, DH), lambda i: (i, 0)),
        ],
        out_shape=[
            jax.ShapeDtypeStruct((N_NODES, DH), jnp.float32),
            jax.ShapeDtypeStruct((N_NODES, DH), jnp.float32),
        ],
    )(pa, ya, yb, dinv, b1, w2)


def _final_body(pa_ref, y2a_ref, y2b_ref, dinv_ref, b2_ref, o_ref):
    pa = pa_ref[...]
    acc = jnp.concatenate(
        [pa[0, 0] + pa[0, 1] + y2a_ref[...],
         pa[1, 0] + pa[1, 1] + y2b_ref[...]], axis=1)
    o_ref[...] = acc * dinv_ref[...] + b2_ref[...]


def _final(pa, y2a, y2b, dinv, b2):
    return pl.pallas_call(
        _final_body,
        grid=(_GRID,),
        in_specs=[
            pl.BlockSpec((2, 2, _RB, DH), lambda i: (0, 0, i, 0)),
            pl.BlockSpec((_RB, DH), lambda i: (i, 0)),
            pl.BlockSpec((_RB, DH), lambda i: (i, 0)),
            pl.BlockSpec((_RB, 1), lambda i: (i, 0)),
            pl.BlockSpec((1, D), lambda i: (0, 0)),
        ],
        out_specs=pl.BlockSpec((_RB, D), lambda i: (i, 0)),
        out_shape=jax.ShapeDtypeStruct((N_NODES, D), jnp.float32),
    )(pa, y2a, y2b, dinv, b2)


def kernel(x, edge_index, W1, b1, W2, b2):
    ei = edge_index.astype(jnp.int32)
    pad = NE_PAD - N_EDGES
    src = jnp.concatenate([ei[0], jnp.zeros((pad,), jnp.int32)])
    junk = N_NODES + (jnp.arange(pad, dtype=jnp.int32) % (N_PAD - N_NODES))
    dst = jnp.concatenate([ei[1], junk])
    src_r = src.reshape(NW, NCH, K)
    dst_r = dst.reshape(NW, NCH, K)

    degp = _deg_call(dst_r)
    dinv, y1a, y1b = _scale(degp, x, W1)

    pa1 = _edge_call(y1a, y1b, src_r, dst_r)
    y2a, y2b = _mid(pa1, y1a, y1b, dinv, b1.reshape(1, D), W2)
    pa2 = _edge_call(y2a, y2b, src_r, dst_r)
    return _final(pa2, y2a, y2b, dinv, b2.reshape(1, D))
